# Initial kernel scaffold; baseline (speedup 1.0000x reference)
import functools

import jax
import jax.numpy as jnp
from jax import lax
from jax.experimental import pallas as pl
from jax.experimental.pallas import tpu as pltpu
from jax.experimental.pallas import tpu_sc as plsc

VOCAB = 100000
D = 128
NUM_CTX = 5
B = 4096
NW = 32
B_PER_W = B // NW
L = 16
NCHUNK = D // L


def _sc_body(tgt_idx_hbm, ctx_idx_hbm, wt_hbm, wc_hbm, out_hbm,
             idx_t, idx_c, te, ce, out_v, sem):
    cid = lax.axis_index("c")
    sid = lax.axis_index("s")
    wid = sid * 2 + cid
    base = wid * B_PER_W

    pltpu.sync_copy(tgt_idx_hbm.at[pl.ds(base, B_PER_W)], idx_t)
    pltpu.sync_copy(ctx_idx_hbm.at[wid], idx_c)

    copies = [pltpu.make_async_copy(wt_hbm.at[idx_t], te, sem)]
    for c in range(NUM_CTX):
        copies.append(
            pltpu.make_async_copy(
                wc_hbm.at[idx_c.at[c]],
                ce.at[pl.ds(c * B_PER_W, B_PER_W)],
                sem,
            )
        )
    for cp in copies:
        cp.start()
    for cp in copies:
        cp.wait()

    def body(b, carry):
        tch = [te[b, pl.ds(k * L, L)] for k in range(NCHUNK)]
        for j in range(NUM_CTX):
            row = b * NUM_CTX + j
            acc = ce[row, pl.ds(0, L)] * tch[0]
            for k in range(1, NCHUNK):
                acc = acc + ce[row, pl.ds(k * L, L)] * tch[k]
            out_v[row] = jnp.sum(acc)
        return carry

    lax.fori_loop(0, B_PER_W, body, 0)

    pltpu.sync_copy(out_v, out_hbm.at[pl.ds(base * NUM_CTX, B_PER_W * NUM_CTX)])


@jax.jit
def _act2vec_sc(tgt_idx, ctx_idx, W_target, W_context):
    mesh = plsc.VectorSubcoreMesh(core_axis_name="c", subcore_axis_name="s")
    kern = functools.partial(
        pl.kernel,
        mesh=mesh,
        out_type=jax.ShapeDtypeStruct((B * NUM_CTX,), jnp.float32),
        scratch_types=[
            pltpu.VMEM((B_PER_W,), jnp.int32),
            pltpu.VMEM((NUM_CTX, B_PER_W), jnp.int32),
            pltpu.VMEM((B_PER_W, D), jnp.float32),
            pltpu.VMEM((NUM_CTX * B_PER_W, D), jnp.float32),
            pltpu.VMEM((B_PER_W * NUM_CTX,), jnp.float32),
            pltpu.SemaphoreType.DMA,
        ],
    )(_sc_body)
    return kern(tgt_idx, ctx_idx, W_target, W_context)


def kernel(target, context, W_target, W_context):
    tgt_idx = target.reshape(B).astype(jnp.int32)
    ctx_idx = context.reshape(NW, NUM_CTX, B_PER_W).astype(jnp.int32)
    out = _act2vec_sc(tgt_idx, ctx_idx, W_target, W_context)
    return out.reshape(B, NUM_CTX)


# speedup vs baseline: 3.0978x; 3.0978x over previous
import functools

import jax
import jax.numpy as jnp
from jax import lax
from jax.experimental import pallas as pl
from jax.experimental.pallas import tpu as pltpu
from jax.experimental.pallas import tpu_sc as plsc

VOCAB = 100000
D = 128
NUM_CTX = 5
B = 4096
NW = 32
B_PER_W = B // NW
L = 16
NCHUNK = D // L


def _sc_body(tgt_idx_hbm, ctx_idx_hbm, wt_hbm, wc_hbm, out_hbm,
             idx_t, idx_c, te, ce, out_v, sem):
    cid = lax.axis_index("c")
    sid = lax.axis_index("s")
    wid = sid * 2 + cid
    base = wid * B_PER_W

    pltpu.sync_copy(tgt_idx_hbm.at[pl.ds(base, B_PER_W)], idx_t)
    pltpu.sync_copy(ctx_idx_hbm.at[wid], idx_c)

    copies = [pltpu.make_async_copy(wt_hbm.at[idx_t], te, sem)]
    for c in range(NUM_CTX):
        copies.append(
            pltpu.make_async_copy(
                wc_hbm.at[idx_c.at[c]],
                ce.at[pl.ds(c * B_PER_W, B_PER_W)],
                sem,
            )
        )
    for cp in copies:
        cp.start()
    for cp in copies:
        cp.wait()

    lanes = lax.iota(jnp.int32, L)

    def body(it, carry):
        res = [jnp.zeros((L,), jnp.float32) for _ in range(NUM_CTX)]
        for i in range(L):
            b = it * L + i
            tch = [te[b, pl.ds(k * L, L)] for k in range(NCHUNK)]
            for j in range(NUM_CTX):
                p = i * NUM_CTX + j
                row = b * NUM_CTX + j
                acc = ce[row, pl.ds(0, L)] * tch[0]
                for k in range(1, NCHUNK):
                    acc = acc + ce[row, pl.ds(k * L, L)] * tch[k]
                r = jnp.sum(acc)
                q, lane = p // L, p % L
                res[q] = jnp.where(lanes == lane, r, res[q])
        for q in range(NUM_CTX):
            out_v[pl.ds(it * (L * NUM_CTX) + q * L, L)] = res[q]
        return carry

    lax.fori_loop(0, B_PER_W // L, body, 0)

    pltpu.sync_copy(out_v, out_hbm.at[pl.ds(base * NUM_CTX, B_PER_W * NUM_CTX)])


@jax.jit
def _act2vec_sc(tgt_idx, ctx_idx, W_target, W_context):
    mesh = plsc.VectorSubcoreMesh(core_axis_name="c", subcore_axis_name="s")
    kern = functools.partial(
        pl.kernel,
        mesh=mesh,
        out_type=jax.ShapeDtypeStruct((B * NUM_CTX,), jnp.float32),
        scratch_types=[
            pltpu.VMEM((B_PER_W,), jnp.int32),
            pltpu.VMEM((NUM_CTX, B_PER_W), jnp.int32),
            pltpu.VMEM((B_PER_W, D), jnp.float32),
            pltpu.VMEM((NUM_CTX * B_PER_W, D), jnp.float32),
            pltpu.VMEM((B_PER_W * NUM_CTX,), jnp.float32),
            pltpu.SemaphoreType.DMA,
        ],
        compiler_params=pltpu.CompilerParams(needs_layout_passes=False),
    )(_sc_body)
    return kern(tgt_idx, ctx_idx, W_target, W_context)


def kernel(target, context, W_target, W_context):
    tgt_idx = target.reshape(B).astype(jnp.int32)
    ctx_idx = context.reshape(NW, NUM_CTX, B_PER_W).astype(jnp.int32)
    out = _act2vec_sc(tgt_idx, ctx_idx, W_target, W_context)
    return out.reshape(B, NUM_CTX)
